# baseline (speedup 1.0000x reference)
import functools

import jax
import jax.numpy as jnp
from jax import lax
from jax.experimental import pallas as pl
from jax.experimental.pallas import tpu as pltpu
from jax.experimental.pallas import tpu_sc as plsc

_NE = 8192
_D = 256
_NV = 16384

_BZ = 8192
_NZB = _NV // _BZ


_LANES = 128
_KS = _NE // _LANES
_BR = 256
_NRB = _BZ // _BR


def _dist_argmin_body(a2_ref, b2_ref, zb_ref, cbt_ref, idx_ref, loss_ref):
    i = pl.program_id(0)
    b2 = b2_ref[...]
    cbt = cbt_ref[...]
    total = None
    for r in range(_NRB):
        rlo, rhi = r * _BR, (r + 1) * _BR
        ab2 = lax.dot_general(
            zb_ref[rlo:rhi, :] * 2.0, cbt, (((1,), (0,)), ((), ())),
            preferred_element_type=jnp.float32)
        a2r = a2_ref[rlo:rhi, :]
        pm = jnp.full((_BR, _LANES), jnp.inf, jnp.float32)
        pb = jnp.zeros((_BR, _LANES), jnp.int32)
        for k in range(_KS):
            lo, hi = k * _LANES, (k + 1) * _LANES
            t1 = a2r + b2[lo:hi][None, :]
            d2 = t1 - ab2[:, lo:hi]
            blk = jnp.full((_BR, _LANES), k, jnp.int32)
            better = d2 < pm
            pb = jnp.where(better, blk, pb)
            pm = jnp.minimum(d2, pm)
        gidx = pb * _LANES + lax.broadcasted_iota(
            jnp.int32, (_BR, _LANES), 1)
        m = jnp.min(pm, axis=1)
        amin = jnp.min(jnp.where(pm == m[:, None], gidx, _NE), axis=1)
        idx_ref[rlo:rhi] = amin
        s = jnp.sum(m)
        total = s if total is None else total + s
    prev = jnp.where(i == 0, 0.0, loss_ref[0, 0])
    loss_ref[0, 0] = prev + total


_dist_argmin = pl.pallas_call(
    _dist_argmin_body,
    grid=(_NZB,),
    in_specs=[
        pl.BlockSpec((_BZ, 1), lambda i: (i, 0)),
        pl.BlockSpec((_NE,), lambda i: (0,)),
        pl.BlockSpec((_BZ, _D), lambda i: (i, 0)),
        pl.BlockSpec((_D, _NE), lambda i: (0, 0)),
    ],
    out_specs=[
        pl.BlockSpec((_BZ,), lambda i: (i,)),
        pl.BlockSpec(memory_space=pltpu.SMEM, block_shape=(1, 1),
                     index_map=lambda i: (0, 0)),
    ],
    out_shape=[
        jax.ShapeDtypeStruct((_NV,), jnp.int32),
        jax.ShapeDtypeStruct((1, 1), jnp.float32),
    ],
)


_NW = 32
_CH = 128

@functools.cache
def _make_gather_rows(nv):
    bpw = nv // _NW
    nch = bpw // _CH
    nbuf = min(3, nch)
    mesh = plsc.VectorSubcoreMesh(core_axis_name="c", subcore_axis_name="s")

    @functools.partial(
        pl.kernel,
        mesh=mesh,
        out_type=jax.ShapeDtypeStruct((nv, _D), jnp.float32),
        scratch_types=(
            [pltpu.VMEM((_CH,), jnp.int32) for _ in range(nch)]
            + [pltpu.VMEM((_CH, _D), jnp.float32) for _ in range(nbuf)]
            + [pltpu.SemaphoreType.DMA for _ in range(2 * nch)]
        ),
    )
    def _gather_rows(cb_hbm, idx_hbm, out_hbm, *s):
        ib = s[:nch]
        rb = s[nch:nch + nbuf]
        gs = s[nch + nbuf:2 * nch + nbuf]
        os_ = s[2 * nch + nbuf:]
        wid = lax.axis_index("s") * 2 + lax.axis_index("c")
        base = wid * bpw
        for ci in range(nch):
            pltpu.sync_copy(idx_hbm.at[pl.ds(base + ci * _CH, _CH)], ib[ci])
        g = [None] * nch
        o = [None] * nch
        for ci in range(nbuf):
            g[ci] = pltpu.async_copy(cb_hbm.at[ib[ci]], rb[ci], gs[ci])
        for ci in range(nch):
            g[ci].wait()
            o[ci] = pltpu.async_copy(
                rb[ci % nbuf], out_hbm.at[pl.ds(base + ci * _CH, _CH)],
                os_[ci])
            nx = ci + nbuf
            if nx < nch:
                o[ci].wait()
                g[nx] = pltpu.async_copy(cb_hbm.at[ib[nx]], rb[ci % nbuf],
                                         gs[nx])
        for ci in range(max(0, nch - nbuf), nch):
            o[ci].wait()

    return _gather_rows


def kernel(z_feats, codebook):
    a2 = jnp.sum(z_feats * z_feats, axis=1)
    b2 = jnp.sum(codebook * codebook, axis=1)
    idx, loss_sum = _dist_argmin(a2[:, None], b2, z_feats, codebook.T)
    quantized = _make_gather_rows(_NV)(codebook, idx)
    loss = loss_sum[0, 0] / jnp.float32(_NV * _D)
    enc = idx.astype(jnp.float32)[:, None]
    return quantized, loss, loss, enc


# speedup vs baseline: 1.2227x; 1.2227x over previous
import functools

import jax
import jax.numpy as jnp
from jax import lax
from jax.experimental import pallas as pl
from jax.experimental.pallas import tpu as pltpu
from jax.experimental.pallas import tpu_sc as plsc

_NE = 8192
_D = 256
_NV = 16384

_BZ = 4096
_NZB = _NV // _BZ


_LANES = 128
_KS = _NE // _LANES
_BR = 512
_NRB = _BZ // _BR


def _dist_argmin_body(a2_ref, b2_ref, zb_ref, cbt_ref, idx_ref, loss_ref):
    i = pl.program_id(0)
    b2 = b2_ref[...]
    cbt = cbt_ref[...]
    total = None
    for r in range(_NRB):
        rlo, rhi = r * _BR, (r + 1) * _BR
        ab2 = lax.dot_general(
            zb_ref[rlo:rhi, :] * 2.0, cbt, (((1,), (0,)), ((), ())),
            preferred_element_type=jnp.float32)
        a2r = a2_ref[rlo:rhi, :]
        pm = jnp.full((_BR, _LANES), jnp.inf, jnp.float32)
        pb = jnp.zeros((_BR, _LANES), jnp.int32)
        for k in range(_KS):
            lo, hi = k * _LANES, (k + 1) * _LANES
            t1 = a2r + b2[lo:hi][None, :]
            d2 = t1 - ab2[:, lo:hi]
            blk = jnp.full((_BR, _LANES), k, jnp.int32)
            better = d2 < pm
            pb = jnp.where(better, blk, pb)
            pm = jnp.minimum(d2, pm)
        gidx = pb * _LANES + lax.broadcasted_iota(
            jnp.int32, (_BR, _LANES), 1)
        m = jnp.min(pm, axis=1)
        amin = jnp.min(jnp.where(pm == m[:, None], gidx, _NE), axis=1)
        idx_ref[rlo:rhi] = amin
        s = jnp.sum(m)
        total = s if total is None else total + s
    prev = jnp.where(i == 0, 0.0, loss_ref[0, 0])
    loss_ref[0, 0] = prev + total


_dist_argmin = pl.pallas_call(
    _dist_argmin_body,
    grid=(_NZB,),
    in_specs=[
        pl.BlockSpec((_BZ, 1), lambda i: (i, 0)),
        pl.BlockSpec((_NE,), lambda i: (0,)),
        pl.BlockSpec((_BZ, _D), lambda i: (i, 0)),
        pl.BlockSpec((_D, _NE), lambda i: (0, 0)),
    ],
    out_specs=[
        pl.BlockSpec((_BZ,), lambda i: (i,)),
        pl.BlockSpec(memory_space=pltpu.SMEM, block_shape=(1, 1),
                     index_map=lambda i: (0, 0)),
    ],
    out_shape=[
        jax.ShapeDtypeStruct((_NV,), jnp.int32),
        jax.ShapeDtypeStruct((1, 1), jnp.float32),
    ],
)


_NW = 32
_CH = 128

@functools.cache
def _make_gather_rows(nv):
    bpw = nv // _NW
    nch = bpw // _CH
    nbuf = min(3, nch)
    mesh = plsc.VectorSubcoreMesh(core_axis_name="c", subcore_axis_name="s")

    @functools.partial(
        pl.kernel,
        mesh=mesh,
        out_type=jax.ShapeDtypeStruct((nv, _D), jnp.float32),
        scratch_types=(
            [pltpu.VMEM((_CH,), jnp.int32) for _ in range(nch)]
            + [pltpu.VMEM((_CH, _D), jnp.float32) for _ in range(nbuf)]
            + [pltpu.SemaphoreType.DMA for _ in range(2 * nch)]
        ),
    )
    def _gather_rows(cb_hbm, idx_hbm, out_hbm, *s):
        ib = s[:nch]
        rb = s[nch:nch + nbuf]
        gs = s[nch + nbuf:2 * nch + nbuf]
        os_ = s[2 * nch + nbuf:]
        wid = lax.axis_index("s") * 2 + lax.axis_index("c")
        base = wid * bpw
        for ci in range(nch):
            pltpu.sync_copy(idx_hbm.at[pl.ds(base + ci * _CH, _CH)], ib[ci])
        g = [None] * nch
        o = [None] * nch
        for ci in range(nbuf):
            g[ci] = pltpu.async_copy(cb_hbm.at[ib[ci]], rb[ci], gs[ci])
        for ci in range(nch):
            g[ci].wait()
            o[ci] = pltpu.async_copy(
                rb[ci % nbuf], out_hbm.at[pl.ds(base + ci * _CH, _CH)],
                os_[ci])
            nx = ci + nbuf
            if nx < nch:
                o[ci].wait()
                g[nx] = pltpu.async_copy(cb_hbm.at[ib[nx]], rb[ci % nbuf],
                                         gs[nx])
        for ci in range(max(0, nch - nbuf), nch):
            o[ci].wait()

    return _gather_rows


def kernel(z_feats, codebook):
    a2 = jnp.sum(z_feats * z_feats, axis=1)
    b2 = jnp.sum(codebook * codebook, axis=1)
    idx, loss_sum = _dist_argmin(a2[:, None], b2, z_feats, codebook.T)
    quantized = _make_gather_rows(_NV)(codebook, idx)
    loss = loss_sum[0, 0] / jnp.float32(_NV * _D)
    enc = idx.astype(jnp.float32)[:, None]
    return quantized, loss, loss, enc
